# Initial kernel scaffold; baseline (speedup 1.0000x reference)
import jax
import jax.numpy as jnp
from jax.experimental import pallas as pl


def kernel(input_ids, table):
    raise NotImplementedError("write your pallas kernel here")


# speedup vs baseline: 1.2890x; 1.2890x over previous
import functools

import jax
import jax.numpy as jnp
from jax.experimental import pallas as pl
from jax.experimental.pallas import tpu as pltpu
from jax.experimental.pallas import tpu_sc as plsc

_W = 64


def _gather_rows(table, idx_flat):
    B = idx_flat.shape[0]
    V, D = table.shape
    mesh = plsc.VectorSubcoreMesh(core_axis_name="c", subcore_axis_name="s")

    @functools.partial(
        pl.kernel,
        mesh=mesh,
        out_type=jax.ShapeDtypeStruct((B, D), table.dtype),
    )
    def emb(table_hbm, idx_hbm, out_hbm):
        def body(i_vmem, o_vmem):
            pltpu.sync_copy(table_hbm.at[i_vmem], o_vmem)

        pltpu.emit_pipeline(
            body,
            grid=(B // _W,),
            in_specs=[pl.BlockSpec((_W,), lambda i: (i,))],
            out_specs=[pl.BlockSpec((_W, D), lambda i: (i, 0))],
            core_axis_name=("c", "s"),
            dimension_semantics=(pltpu.PARALLEL,),
        )(idx_hbm, out_hbm)

    return emb(table, idx_flat)


def kernel(input_ids, table):
    Bb, S = input_ids.shape
    D = table.shape[1]
    out = _gather_rows(table, input_ids.reshape(Bb * S))
    return out.reshape(Bb, S, D)


# speedup vs baseline: 3.8275x; 2.9693x over previous
import functools

import jax
import jax.numpy as jnp
from jax.experimental import pallas as pl
from jax.experimental.pallas import tpu as pltpu
from jax.experimental.pallas import tpu_sc as plsc

_W = 64


def _gather_rows(table, idx_flat):
    B = idx_flat.shape[0]
    V, D = table.shape
    mesh = plsc.VectorSubcoreMesh(core_axis_name="c", subcore_axis_name="s")

    @functools.partial(
        pl.kernel,
        mesh=mesh,
        out_type=jax.ShapeDtypeStruct((B, D), table.dtype),
    )
    def emb(table_hbm, idx_hbm, out_hbm):
        def body(i_vmem, o_vmem):
            pltpu.sync_copy(table_hbm.at[i_vmem], o_vmem)

        pltpu.emit_pipeline(
            body,
            grid=(B // _W,),
            in_specs=[pl.BlockSpec((_W,), lambda i: (i,))],
            out_specs=[pl.BlockSpec((_W, D), lambda i: (i, 0))],
            core_axis_name=("c", "s"),
            dimension_semantics=(pltpu.PARALLEL,),
        )(idx_hbm, out_hbm)

    return emb(table, idx_flat)


def kernel(input_ids, table):
    Bb, S = input_ids.shape
    D = table.shape[1]
    out = _gather_rows(table, input_ids.T.reshape(Bb * S))
    return out.reshape(S, Bb, D).transpose(1, 0, 2)
